# baseline (speedup 1.0000x reference)
import functools

import jax
import jax.numpy as jnp
from jax import lax
from jax.experimental import pallas as pl
from jax.experimental.pallas import tpu as pltpu
from jax.experimental.pallas import tpu_sc as plsc

_H = 4
_C = 32
_HID = 128
_NA = 100000
_NC = 50000
_E = 600000
_B = 2048

_NSC = 2
_NSUB = 16
_K = 8448
_NCHUNK = 6
_CPS = 3
_DUMP = _K
_ACCROWS = _K + 16
_EBATCH = 64
_EB2 = 128
_EPAD = 602112
_NBATCH = _EPAD // (_NSUB * _EBATCH)
_STRIPE = _K // _NSUB

_R = 512
_NCPAD = 50688
_G = _NCPAD // _R


def _s1_body(x_hbm, row_hbm, col_hbm, sums_hbm, acc,
             colbuf0, gidx0, sidx0, rowbuf0, colbuf1, gidx1, sidx1, rowbuf1,
             colbuf2, gidx2, sidx2, rowbuf2, colbuf3, gidx3, sidx3, rowbuf3,
             sem0, sem1, sem2, sem3):
    c = lax.axis_index("c")
    s = lax.axis_index("s")
    zv = jnp.zeros((16,), jnp.float32)
    r0 = s * _STRIPE
    ngrp = _HID // 16
    colbufs = [colbuf0, colbuf1, colbuf2, colbuf3]
    gidxs = [gidx0, gidx1, gidx2, gidx3]
    sidxs = [sidx0, sidx1, sidx2, sidx3]
    rowbufs = [rowbuf0, rowbuf1, rowbuf2, rowbuf3]
    sems = [sem0, sem1, sem2, sem3]
    rowbuf = rowbuf0

    for k in range(_CPS):
        chunk = _CPS * c + k
        base = chunk * _K

        def _zb(i, _):
            for j in range(ngrp):
                rowbuf[i, pl.ds(16 * j, 16)] = zv
            return 0

        lax.fori_loop(0, _EBATCH, _zb, 0)

        def _z(i, _):
            pltpu.sync_copy(rowbuf, acc.at[pl.ds(r0 + i * _EBATCH, _EBATCH)])
            return 0

        lax.fori_loop(0, 512 // _EBATCH, _z, 0)
        pltpu.sync_copy(rowbuf.at[pl.ds(0, 16)], acc.at[pl.ds(r0 + 512, 16)])

        @pl.when(s == _NSUB - 1)
        def _zero_dump():
            pltpu.sync_copy(rowbuf.at[pl.ds(0, 16)], acc.at[pl.ds(_K, 16)])

        plsc.subcore_barrier()

        ebase = s * (_NBATCH * _EBATCH)

        def _sidx_of(cb, si):
            for j in range(_EBATCH // 16):
                v = cb[pl.ds(16 * j, 16)]
                l = v - base
                ok = (l >= 0) & (l < _K)
                si[pl.ds(16 * j, 16)] = jnp.where(ok, l, _DUMP)

        def _quad(i, _):
            descs = []
            for q in range(4):
                off = ebase + (4 * i + q) * _EBATCH
                pltpu.sync_copy(row_hbm.at[pl.ds(off, _EBATCH)], gidxs[q])
                pltpu.sync_copy(col_hbm.at[pl.ds(off, _EBATCH)], colbufs[q])
                descs.append(
                    pltpu.async_copy(x_hbm.at[gidxs[q]], rowbufs[q], sems[q]))
            for q in range(4):
                _sidx_of(colbufs[q], sidxs[q])
            for q in range(4):
                descs[q].wait()
                pltpu.sync_copy(rowbufs[q], acc.at[sidxs[q]], add=True)
            return 0

        lax.fori_loop(0, _NBATCH // 4, _quad, 0)
        plsc.subcore_barrier()

        def _eblk(i, _):
            pltpu.sync_copy(acc.at[pl.ds(r0 + i * 128, 128)],
                            sums_hbm.at[pl.ds(base + r0 + i * 128, 128)])
            return 0

        lax.fori_loop(0, 4, _eblk, 0)
        pltpu.sync_copy(acc.at[pl.ds(r0 + 512, 16)],
                        sums_hbm.at[pl.ds(base + r0 + 512, 16)])
        plsc.subcore_barrier()


@jax.jit
def _sc_sums(x, rowp, colp):
    mesh = plsc.VectorSubcoreMesh(core_axis_name="c", subcore_axis_name="s")
    f = pl.kernel(
        _s1_body,
        out_type=jax.ShapeDtypeStruct((_NCHUNK * _K, _HID), jnp.float32),
        mesh=mesh,
        scratch_types=(
            [pltpu.VMEM_SHARED((_ACCROWS, _HID), jnp.float32)] +
            [t for _q in range(4) for t in
             (pltpu.VMEM((_EBATCH,), jnp.int32),
              pltpu.VMEM((_EBATCH,), jnp.int32),
              pltpu.VMEM((_EBATCH,), jnp.int32),
              pltpu.VMEM((_EBATCH, _HID), jnp.float32))] +
            [pltpu.SemaphoreType.DMA] * 4
        ),
    )
    return f(x, rowp, colp)


_CROWS = 512
_NB2 = _EPAD // (_NSC * _NSUB * _EB2)


def _s2_body(col_hbm, cnt_hbm, acc2, colbuf, sidx, onebuf):
    c = lax.axis_index("c")
    s = lax.axis_index("s")
    w = c * _NSUB + s
    zv = jnp.zeros((16,), jnp.float32)
    ngrp = _HID // 16

    def _zb(i, _):
        for j in range(ngrp):
            onebuf[i, pl.ds(16 * j, 16)] = zv
        return 0

    lax.fori_loop(0, _EB2, _zb, 0)
    pltpu.sync_copy(onebuf.at[pl.ds(0, 32)], acc2.at[pl.ds(s * 32, 32)])
    plsc.subcore_barrier()

    lane = lax.iota(jnp.int32, 16)
    ebase = w * (_NB2 * _EB2)

    def _batch(b, _):
        off = ebase + b * _EB2
        pltpu.sync_copy(col_hbm.at[pl.ds(off, _EB2)], colbuf)
        for j in range(_EB2 // 16):
            v = colbuf[pl.ds(16 * j, 16)]
            sidx[pl.ds(16 * j, 16)] = jnp.minimum(
                lax.shift_right_logical(v, 7), _CROWS - 1)
            low = v & 127
            for r in range(16):
                lr = jnp.broadcast_to(low[r], (16,))
                for g in range(ngrp):
                    onebuf[16 * j + r, pl.ds(16 * g, 16)] = jnp.where(
                        lane + 16 * g == lr, 1.0, 0.0)
        pltpu.sync_copy(onebuf, acc2.at[sidx], add=True)
        return 0

    lax.fori_loop(0, _NB2, _batch, 0)
    plsc.subcore_barrier()

    pltpu.sync_copy(acc2.at[pl.ds(s * 32, 32)],
                    cnt_hbm.at[pl.ds(c * _CROWS + s * 32, 32)])


@jax.jit
def _sc_counts(colp):
    mesh = plsc.VectorSubcoreMesh(core_axis_name="c", subcore_axis_name="s")
    f = pl.kernel(
        _s2_body,
        out_type=jax.ShapeDtypeStruct((_NSC * _CROWS, _HID), jnp.float32),
        mesh=mesh,
        scratch_types=[
            pltpu.VMEM_SHARED((_CROWS, _HID), jnp.float32),
            pltpu.VMEM((_EB2,), jnp.int32),
            pltpu.VMEM((_EB2,), jnp.int32),
            pltpu.VMEM((_EB2, _HID), jnp.float32),
        ],
    )
    return f(colp)


def _ka_body(sums_ref, cnt_ref, xcl_ref, wp_ref, bp_ref, w1_ref, b1_ref,
             w2_ref, b2_ref, xc_ref, sc_ref):
    pid = pl.program_id(0)
    hx = sums_ref[...] / jnp.maximum(cnt_ref[...], 1.0)
    p = jnp.dot(hx, wp_ref[...], preferred_element_type=jnp.float32)
    p = p + bp_ref[0:1, :]
    xc = xcl_ref[...] + jnp.maximum(p, 0.0)
    xc_ref[...] = xc
    hm = jnp.dot(xc, w1_ref[...], preferred_element_type=jnp.float32)
    hm = jnp.maximum(hm + b1_ref[0:1, :], 0.0)
    sc = jnp.dot(hm, w2_ref[...], preferred_element_type=jnp.float32)
    sc = sc[:, 0:_H] + b2_ref[0:1, 0:_H]
    rows = pid * _R + lax.broadcasted_iota(jnp.int32, (_R, 1), 0)
    sc_ref[...] = jnp.where(rows < _NC, sc, -1e30)


def _kb1_body(sc_ref, ids_ref, out_ref):
    @pl.when(pl.program_id(0) == 0)
    def _():
        out_ref[...] = jnp.full((8, _B), -1e30, jnp.float32)

    ids = ids_ref[0, 0, :]
    mask = ids[:, None] == lax.broadcasted_iota(jnp.int32, (_R, _B), 1)
    ms = []
    for h in range(_H):
        col = jnp.where(mask, sc_ref[:, h][:, None], -1e30)
        ms.append(jnp.max(col, axis=0)[None, :])
    m4 = jnp.concatenate(ms, axis=0)
    out_ref[0:_H, :] = jnp.maximum(out_ref[0:_H, :], m4)


def _kb2_body(sc_ref, ids_ref, smax_ref, ex_ref, ssum_ref):
    @pl.when(pl.program_id(0) == 0)
    def _():
        ssum_ref[...] = jnp.zeros((8, _B), jnp.float32)

    ids = ids_ref[0, 0, :]
    onehot = (ids[:, None] ==
              lax.broadcasted_iota(jnp.int32, (_R, _B), 1)).astype(jnp.float32)
    rowmax = lax.dot_general(onehot, smax_ref[0:_H, :],
                             (((1,), (1,)), ((), ())),
                             preferred_element_type=jnp.float32)
    ex = jnp.exp(sc_ref[...] - rowmax)
    ex_ref[...] = ex
    contrib = lax.dot_general(ex, onehot, (((0,), (0,)), ((), ())),
                              preferred_element_type=jnp.float32)
    ssum_ref[0:_H, :] += contrib


def _kc_body(ex_ref, ids_ref, ssum_ref, xc_ref, rep_ref, al_ref, df_ref):
    pid = pl.program_id(0)

    @pl.when(pid == 0)
    def _():
        df_ref[...] = jnp.zeros((_B, _HID), jnp.float32)

    ids = ids_ref[0, 0, :]
    onehot = (ids[:, None] ==
              lax.broadcasted_iota(jnp.int32, (_R, _B), 1)).astype(jnp.float32)
    rowsum = lax.dot_general(onehot, ssum_ref[0:_H, :],
                             (((1,), (1,)), ((), ())),
                             preferred_element_type=jnp.float32)
    alpha = ex_ref[...] / (rowsum + 1e-16)
    al_ref[...] = alpha
    aexp = jnp.dot(alpha, rep_ref[...], preferred_element_type=jnp.float32)
    drug = xc_ref[...] * aexp
    rows = pid * _R + lax.broadcasted_iota(jnp.int32, (_R, 1), 0)
    drug = jnp.where(rows < _NC, drug, 0.0)
    df_ref[...] += lax.dot_general(onehot, drug, (((0,), (0,)), ((), ())),
                                   preferred_element_type=jnp.float32)


def _row_spec(w):
    return pl.BlockSpec((_R, w), lambda i: (i, 0))


def _full_spec(a, b):
    return pl.BlockSpec((a, b), lambda i: (0, 0))


_IDS_SPEC = pl.BlockSpec((1, 1, _R), lambda i: (i, 0, 0))


@jax.jit
def _tc_pipeline(sums, cnt128, xcl, ids_r, Wp, bp8, W1bd, b1f, W2bd, b2f,
                 rep):
    xc, score = pl.pallas_call(
        _ka_body,
        grid=(_G,),
        in_specs=[_row_spec(_HID), _row_spec(_HID), _row_spec(_HID),
                  _full_spec(_HID, _HID),
                  _full_spec(8, _HID), _full_spec(_HID, 2 * _C * _H),
                  _full_spec(8, 2 * _C * _H), _full_spec(2 * _C * _H, _HID),
                  _full_spec(8, _HID)],
        out_specs=[_row_spec(_HID), _row_spec(_H)],
        out_shape=[jax.ShapeDtypeStruct((_NCPAD, _HID), jnp.float32),
                   jax.ShapeDtypeStruct((_NCPAD, _H), jnp.float32)],
    )(sums, cnt128, xcl, Wp, bp8, W1bd, b1f, W2bd, b2f)

    smax = pl.pallas_call(
        _kb1_body,
        grid=(_G,),
        in_specs=[_row_spec(_H), _IDS_SPEC],
        out_specs=_full_spec(8, _B),
        out_shape=jax.ShapeDtypeStruct((8, _B), jnp.float32),
    )(score, ids_r)

    ex, ssum = pl.pallas_call(
        _kb2_body,
        grid=(_G,),
        in_specs=[_row_spec(_H), _IDS_SPEC, _full_spec(8, _B)],
        out_specs=[_row_spec(_H), _full_spec(8, _B)],
        out_shape=[jax.ShapeDtypeStruct((_NCPAD, _H), jnp.float32),
                   jax.ShapeDtypeStruct((8, _B), jnp.float32)],
    )(score, ids_r, smax)

    alpha, drug_feat = pl.pallas_call(
        _kc_body,
        grid=(_G,),
        in_specs=[_row_spec(_H), _IDS_SPEC, _full_spec(8, _B),
                  _row_spec(_HID), _full_spec(_H, _HID)],
        out_specs=[_row_spec(_H), _full_spec(_B, _HID)],
        out_shape=[jax.ShapeDtypeStruct((_NCPAD, _H), jnp.float32),
                   jax.ShapeDtypeStruct((_B, _HID), jnp.float32)],
    )(ex, ids_r, ssum, xc, rep)

    return xc, alpha, drug_feat


def kernel(x, x_clique, atom2clique_index, clique_batch, clique_edge_index,
           Wp, bp, W1, b1, W2, b2):
    del clique_edge_index
    idx = atom2clique_index.astype(jnp.int32)
    rowp = jnp.concatenate(
        [idx[0], jnp.zeros((_EPAD - _E,), jnp.int32)])
    colp = jnp.concatenate(
        [idx[1], jnp.full((_EPAD - _E,), 1 << 20, jnp.int32)])

    sums = _sc_sums(x, rowp, colp)
    cpart = _sc_counts(colp)
    cnt = (cpart[:_CROWS] + cpart[_CROWS:]).reshape(-1)[:_NCPAD]
    cnt128 = jnp.broadcast_to(cnt[:, None], (_NCPAD, _HID))

    xcl = jnp.concatenate(
        [x_clique, jnp.zeros((_NCPAD - _NC, _HID), jnp.float32)])
    ids = jnp.concatenate(
        [clique_batch.astype(jnp.int32),
         jnp.full((_NCPAD - _NC,), _B, jnp.int32)])
    ids_r = ids.reshape(_G, 1, _R)

    W1bd = jnp.zeros((_HID, 2 * _C * _H), jnp.float32)
    W2bd = jnp.zeros((2 * _C * _H, _HID), jnp.float32)
    for h in range(_H):
        W1bd = W1bd.at[h * _C:(h + 1) * _C,
                       h * 2 * _C:(h + 1) * 2 * _C].set(W1[h])
        W2bd = W2bd.at[h * 2 * _C:(h + 1) * 2 * _C, h].set(W2[h, :, 0])
    b1f = jnp.zeros((8, 2 * _C * _H), jnp.float32).at[0].set(b1.reshape(-1))
    bp8 = jnp.zeros((8, _HID), jnp.float32).at[0].set(bp)
    b2f = jnp.zeros((8, _HID), jnp.float32).at[0, 0:_H].set(b2[:, 0])
    rep = (jnp.arange(_H)[:, None] == (jnp.arange(_HID) // _C)[None, :]
           ).astype(jnp.float32)

    xc, alpha, drug_feat = _tc_pipeline(
        sums, cnt128, xcl, ids_r, Wp, bp8, W1bd, b1f, W2bd, b2f, rep)

    return drug_feat, xc[:_NC], alpha[:_NC]


# speedup vs baseline: 1.0646x; 1.0646x over previous
import functools

import jax
import jax.numpy as jnp
from jax import lax
from jax.experimental import pallas as pl
from jax.experimental.pallas import tpu as pltpu
from jax.experimental.pallas import tpu_sc as plsc

_H = 4
_C = 32
_HID = 128
_NA = 100000
_NC = 50000
_E = 600000
_B = 2048

_NSC = 2
_NSUB = 16
_K = 8448
_NCHUNK = 6
_CPS = 3
_DUMP = _K
_ACCROWS = _K + 16
_EBATCH = 96
_EB2 = 128
_EPAD = 602112
_NBATCH = _EPAD // (_NSUB * _EBATCH)
_STRIPE = _K // _NSUB

_R = 512
_NCPAD = 50688
_G = _NCPAD // _R


def _s1_body(x_hbm, row_hbm, col_hbm, sums_hbm, acc,
             colbuf0, gidx0, sidx0, rowbuf0, colbuf1, gidx1, sidx1, rowbuf1,
             colbuf2, gidx2, sidx2, rowbuf2, colbuf3, gidx3, sidx3, rowbuf3,
             sem0, sem1, sem2, sem3):
    c = lax.axis_index("c")
    s = lax.axis_index("s")
    zv = jnp.zeros((16,), jnp.float32)
    r0 = s * _STRIPE
    ngrp = _HID // 16
    colbufs = [colbuf0, colbuf1, colbuf2, colbuf3]
    gidxs = [gidx0, gidx1, gidx2, gidx3]
    sidxs = [sidx0, sidx1, sidx2, sidx3]
    rowbufs = [rowbuf0, rowbuf1, rowbuf2, rowbuf3]
    sems = [sem0, sem1, sem2, sem3]
    rowbuf = rowbuf0

    for k in range(_CPS):
        chunk = _CPS * c + k
        base = chunk * _K

        def _zb(i, _):
            for j in range(ngrp):
                rowbuf[i, pl.ds(16 * j, 16)] = zv
            return 0

        lax.fori_loop(0, _EBATCH, _zb, 0)

        def _z(i, _):
            pltpu.sync_copy(rowbuf, acc.at[pl.ds(r0 + i * 96, 96)])
            return 0

        lax.fori_loop(0, 5, _z, 0)
        pltpu.sync_copy(rowbuf.at[pl.ds(0, 48)], acc.at[pl.ds(r0 + 480, 48)])

        @pl.when(s == _NSUB - 1)
        def _zero_dump():
            pltpu.sync_copy(rowbuf.at[pl.ds(0, 16)], acc.at[pl.ds(_K, 16)])

        plsc.subcore_barrier()

        ebase = s * (_NBATCH * _EBATCH)

        def _sidx_of(cb, si):
            for j in range(_EBATCH // 16):
                v = cb[pl.ds(16 * j, 16)]
                l = v - base
                ok = (l >= 0) & (l < _K)
                si[pl.ds(16 * j, 16)] = jnp.where(ok, l, _DUMP)

        def _quad(i, _):
            descs = []
            for q in range(4):
                off = ebase + (4 * i + q) * _EBATCH
                pltpu.sync_copy(row_hbm.at[pl.ds(off, _EBATCH)], gidxs[q])
                pltpu.sync_copy(col_hbm.at[pl.ds(off, _EBATCH)], colbufs[q])
                descs.append(
                    pltpu.async_copy(x_hbm.at[gidxs[q]], rowbufs[q], sems[q]))
            for q in range(4):
                _sidx_of(colbufs[q], sidxs[q])
            for q in range(4):
                descs[q].wait()
                pltpu.sync_copy(rowbufs[q], acc.at[sidxs[q]], add=True)
            return 0

        lax.fori_loop(0, _NBATCH // 4, _quad, 0)
        plsc.subcore_barrier()

        def _eblk(i, _):
            pltpu.sync_copy(acc.at[pl.ds(r0 + i * 128, 128)],
                            sums_hbm.at[pl.ds(base + r0 + i * 128, 128)])
            return 0

        lax.fori_loop(0, 4, _eblk, 0)
        pltpu.sync_copy(acc.at[pl.ds(r0 + 512, 16)],
                        sums_hbm.at[pl.ds(base + r0 + 512, 16)])
        plsc.subcore_barrier()


@jax.jit
def _sc_sums(x, rowp, colp):
    mesh = plsc.VectorSubcoreMesh(core_axis_name="c", subcore_axis_name="s")
    f = pl.kernel(
        _s1_body,
        out_type=jax.ShapeDtypeStruct((_NCHUNK * _K, _HID), jnp.float32),
        mesh=mesh,
        scratch_types=(
            [pltpu.VMEM_SHARED((_ACCROWS, _HID), jnp.float32)] +
            [t for _q in range(4) for t in
             (pltpu.VMEM((_EBATCH,), jnp.int32),
              pltpu.VMEM((_EBATCH,), jnp.int32),
              pltpu.VMEM((_EBATCH,), jnp.int32),
              pltpu.VMEM((_EBATCH, _HID), jnp.float32))] +
            [pltpu.SemaphoreType.DMA] * 4
        ),
    )
    return f(x, rowp, colp)


_CROWS = 512
_NB2 = _EPAD // (_NSC * _NSUB * _EB2)


def _s2_body(col_hbm, cnt_hbm, acc2, colbuf, sidx, onebuf):
    c = lax.axis_index("c")
    s = lax.axis_index("s")
    w = c * _NSUB + s
    zv = jnp.zeros((16,), jnp.float32)
    ngrp = _HID // 16

    def _zb(i, _):
        for j in range(ngrp):
            onebuf[i, pl.ds(16 * j, 16)] = zv
        return 0

    lax.fori_loop(0, _EB2, _zb, 0)
    pltpu.sync_copy(onebuf.at[pl.ds(0, 32)], acc2.at[pl.ds(s * 32, 32)])
    plsc.subcore_barrier()

    lane = lax.iota(jnp.int32, 16)
    ebase = w * (_NB2 * _EB2)

    def _batch(b, _):
        off = ebase + b * _EB2
        pltpu.sync_copy(col_hbm.at[pl.ds(off, _EB2)], colbuf)
        for j in range(_EB2 // 16):
            v = colbuf[pl.ds(16 * j, 16)]
            sidx[pl.ds(16 * j, 16)] = jnp.minimum(
                lax.shift_right_logical(v, 7), _CROWS - 1)
            low = v & 127
            for r in range(16):
                lr = jnp.broadcast_to(low[r], (16,))
                for g in range(ngrp):
                    onebuf[16 * j + r, pl.ds(16 * g, 16)] = jnp.where(
                        lane + 16 * g == lr, 1.0, 0.0)
        pltpu.sync_copy(onebuf, acc2.at[sidx], add=True)
        return 0

    lax.fori_loop(0, _NB2, _batch, 0)
    plsc.subcore_barrier()

    pltpu.sync_copy(acc2.at[pl.ds(s * 32, 32)],
                    cnt_hbm.at[pl.ds(c * _CROWS + s * 32, 32)])


@jax.jit
def _sc_counts(colp):
    mesh = plsc.VectorSubcoreMesh(core_axis_name="c", subcore_axis_name="s")
    f = pl.kernel(
        _s2_body,
        out_type=jax.ShapeDtypeStruct((_NSC * _CROWS, _HID), jnp.float32),
        mesh=mesh,
        scratch_types=[
            pltpu.VMEM_SHARED((_CROWS, _HID), jnp.float32),
            pltpu.VMEM((_EB2,), jnp.int32),
            pltpu.VMEM((_EB2,), jnp.int32),
            pltpu.VMEM((_EB2, _HID), jnp.float32),
        ],
    )
    return f(colp)


def _ka_body(sums_ref, cnt_ref, xcl_ref, wp_ref, bp_ref, w1_ref, b1_ref,
             w2_ref, b2_ref, xc_ref, sc_ref):
    pid = pl.program_id(0)
    hx = sums_ref[...] / jnp.maximum(cnt_ref[...], 1.0)
    p = jnp.dot(hx, wp_ref[...], preferred_element_type=jnp.float32)
    p = p + bp_ref[0:1, :]
    xc = xcl_ref[...] + jnp.maximum(p, 0.0)
    xc_ref[...] = xc
    hm = jnp.dot(xc, w1_ref[...], preferred_element_type=jnp.float32)
    hm = jnp.maximum(hm + b1_ref[0:1, :], 0.0)
    sc = jnp.dot(hm, w2_ref[...], preferred_element_type=jnp.float32)
    sc = sc[:, 0:_H] + b2_ref[0:1, 0:_H]
    rows = pid * _R + lax.broadcasted_iota(jnp.int32, (_R, 1), 0)
    sc_ref[...] = jnp.where(rows < _NC, sc, -1e30)


def _kb1_body(sc_ref, ids_ref, out_ref):
    @pl.when(pl.program_id(0) == 0)
    def _():
        out_ref[...] = jnp.full((8, _B), -1e30, jnp.float32)

    ids = ids_ref[0, 0, :]
    mask = ids[:, None] == lax.broadcasted_iota(jnp.int32, (_R, _B), 1)
    ms = []
    for h in range(_H):
        col = jnp.where(mask, sc_ref[:, h][:, None], -1e30)
        ms.append(jnp.max(col, axis=0)[None, :])
    m4 = jnp.concatenate(ms, axis=0)
    out_ref[0:_H, :] = jnp.maximum(out_ref[0:_H, :], m4)


def _kb2_body(sc_ref, ids_ref, smax_ref, ex_ref, ssum_ref):
    @pl.when(pl.program_id(0) == 0)
    def _():
        ssum_ref[...] = jnp.zeros((8, _B), jnp.float32)

    ids = ids_ref[0, 0, :]
    onehot = (ids[:, None] ==
              lax.broadcasted_iota(jnp.int32, (_R, _B), 1)).astype(jnp.float32)
    rowmax = lax.dot_general(onehot, smax_ref[0:_H, :],
                             (((1,), (1,)), ((), ())),
                             preferred_element_type=jnp.float32)
    ex = jnp.exp(sc_ref[...] - rowmax)
    ex_ref[...] = ex
    contrib = lax.dot_general(ex, onehot, (((0,), (0,)), ((), ())),
                              preferred_element_type=jnp.float32)
    ssum_ref[0:_H, :] += contrib


def _kc_body(ex_ref, ids_ref, ssum_ref, xc_ref, rep_ref, al_ref, df_ref):
    pid = pl.program_id(0)

    @pl.when(pid == 0)
    def _():
        df_ref[...] = jnp.zeros((_B, _HID), jnp.float32)

    ids = ids_ref[0, 0, :]
    onehot = (ids[:, None] ==
              lax.broadcasted_iota(jnp.int32, (_R, _B), 1)).astype(jnp.float32)
    rowsum = lax.dot_general(onehot, ssum_ref[0:_H, :],
                             (((1,), (1,)), ((), ())),
                             preferred_element_type=jnp.float32)
    alpha = ex_ref[...] / (rowsum + 1e-16)
    al_ref[...] = alpha
    aexp = jnp.dot(alpha, rep_ref[...], preferred_element_type=jnp.float32)
    drug = xc_ref[...] * aexp
    rows = pid * _R + lax.broadcasted_iota(jnp.int32, (_R, 1), 0)
    drug = jnp.where(rows < _NC, drug, 0.0)
    df_ref[...] += lax.dot_general(onehot, drug, (((0,), (0,)), ((), ())),
                                   preferred_element_type=jnp.float32)


def _row_spec(w):
    return pl.BlockSpec((_R, w), lambda i: (i, 0))


def _full_spec(a, b):
    return pl.BlockSpec((a, b), lambda i: (0, 0))


_IDS_SPEC = pl.BlockSpec((1, 1, _R), lambda i: (i, 0, 0))


@jax.jit
def _tc_pipeline(sums, cnt128, xcl, ids_r, Wp, bp8, W1bd, b1f, W2bd, b2f,
                 rep):
    xc, score = pl.pallas_call(
        _ka_body,
        grid=(_G,),
        in_specs=[_row_spec(_HID), _row_spec(_HID), _row_spec(_HID),
                  _full_spec(_HID, _HID),
                  _full_spec(8, _HID), _full_spec(_HID, 2 * _C * _H),
                  _full_spec(8, 2 * _C * _H), _full_spec(2 * _C * _H, _HID),
                  _full_spec(8, _HID)],
        out_specs=[_row_spec(_HID), _row_spec(_H)],
        out_shape=[jax.ShapeDtypeStruct((_NCPAD, _HID), jnp.float32),
                   jax.ShapeDtypeStruct((_NCPAD, _H), jnp.float32)],
    )(sums, cnt128, xcl, Wp, bp8, W1bd, b1f, W2bd, b2f)

    smax = pl.pallas_call(
        _kb1_body,
        grid=(_G,),
        in_specs=[_row_spec(_H), _IDS_SPEC],
        out_specs=_full_spec(8, _B),
        out_shape=jax.ShapeDtypeStruct((8, _B), jnp.float32),
    )(score, ids_r)

    ex, ssum = pl.pallas_call(
        _kb2_body,
        grid=(_G,),
        in_specs=[_row_spec(_H), _IDS_SPEC, _full_spec(8, _B)],
        out_specs=[_row_spec(_H), _full_spec(8, _B)],
        out_shape=[jax.ShapeDtypeStruct((_NCPAD, _H), jnp.float32),
                   jax.ShapeDtypeStruct((8, _B), jnp.float32)],
    )(score, ids_r, smax)

    alpha, drug_feat = pl.pallas_call(
        _kc_body,
        grid=(_G,),
        in_specs=[_row_spec(_H), _IDS_SPEC, _full_spec(8, _B),
                  _row_spec(_HID), _full_spec(_H, _HID)],
        out_specs=[_row_spec(_H), _full_spec(_B, _HID)],
        out_shape=[jax.ShapeDtypeStruct((_NCPAD, _H), jnp.float32),
                   jax.ShapeDtypeStruct((_B, _HID), jnp.float32)],
    )(ex, ids_r, ssum, xc, rep)

    return xc, alpha, drug_feat


def kernel(x, x_clique, atom2clique_index, clique_batch, clique_edge_index,
           Wp, bp, W1, b1, W2, b2):
    del clique_edge_index
    idx = atom2clique_index.astype(jnp.int32)
    rowp = jnp.concatenate(
        [idx[0], jnp.zeros((_EPAD - _E,), jnp.int32)])
    colp = jnp.concatenate(
        [idx[1], jnp.full((_EPAD - _E,), 1 << 20, jnp.int32)])

    sums = _sc_sums(x, rowp, colp)
    cpart = _sc_counts(colp)
    cnt = (cpart[:_CROWS] + cpart[_CROWS:]).reshape(-1)[:_NCPAD]
    cnt128 = jnp.broadcast_to(cnt[:, None], (_NCPAD, _HID))

    xcl = jnp.concatenate(
        [x_clique, jnp.zeros((_NCPAD - _NC, _HID), jnp.float32)])
    ids = jnp.concatenate(
        [clique_batch.astype(jnp.int32),
         jnp.full((_NCPAD - _NC,), _B, jnp.int32)])
    ids_r = ids.reshape(_G, 1, _R)

    W1bd = jnp.zeros((_HID, 2 * _C * _H), jnp.float32)
    W2bd = jnp.zeros((2 * _C * _H, _HID), jnp.float32)
    for h in range(_H):
        W1bd = W1bd.at[h * _C:(h + 1) * _C,
                       h * 2 * _C:(h + 1) * 2 * _C].set(W1[h])
        W2bd = W2bd.at[h * 2 * _C:(h + 1) * 2 * _C, h].set(W2[h, :, 0])
    b1f = jnp.zeros((8, 2 * _C * _H), jnp.float32).at[0].set(b1.reshape(-1))
    bp8 = jnp.zeros((8, _HID), jnp.float32).at[0].set(bp)
    b2f = jnp.zeros((8, _HID), jnp.float32).at[0, 0:_H].set(b2[:, 0])
    rep = (jnp.arange(_H)[:, None] == (jnp.arange(_HID) // _C)[None, :]
           ).astype(jnp.float32)

    xc, alpha, drug_feat = _tc_pipeline(
        sums, cnt128, xcl, ids_r, Wp, bp8, W1bd, b1f, W2bd, b2f, rep)

    return drug_feat, xc[:_NC], alpha[:_NC]
